# Initial kernel scaffold; baseline (speedup 1.0000x reference)
import jax
import jax.numpy as jnp
from jax.experimental import pallas as pl


def kernel(region_id, latitude, longitude, arrival_time, departure_time, poly_W_lin, poly_b_lin, poly_W_per, poly_b_per, t2v_w0, t2v_b0, t2v_W, t2v_B, table):
    raise NotImplementedError("write your pallas kernel here")


# speedup vs baseline: 1.3423x; 1.3423x over previous
import functools

import jax
import jax.numpy as jnp
from jax import lax
from jax.experimental import pallas as pl
from jax.experimental.pallas import tpu as pltpu
from jax.experimental.pallas import tpu_sc as plsc

_PAD = 0
_GW = 128
_ROWS_BLK = 1024


def _sc_gather(table, idx2d, n, d):
    mesh = plsc.VectorSubcoreMesh(core_axis_name="core",
                                  subcore_axis_name="subcore")

    @functools.partial(
        pl.kernel,
        out_type=jax.ShapeDtypeStruct((n, d), table.dtype),
        mesh=mesh,
        compiler_params=pltpu.CompilerParams(use_tc_tiling_on_sc=False),
    )
    def k(tbl_hbm, i_hbm, o_hbm):
        def body(i_vmem, o_vmem):
            pltpu.sync_copy(tbl_hbm.at[i_vmem.at[0]], o_vmem)

        pltpu.emit_pipeline(
            body,
            grid=(n // _GW,),
            in_specs=[pl.BlockSpec((1, _GW), index_map=lambda i: (0, i))],
            out_specs=[pl.BlockSpec((_GW, d), index_map=lambda i: (i, 0))],
            core_axis_name=("core", "subcore"),
            dimension_semantics=(pltpu.PARALLEL,),
        )(i_hbm, o_hbm)

    return k(table, idx2d)


def _tc_encode_kernel(lat_ref, lon_ref, ta_ref, td_ref, rows_ref,
                      wa_ref, wb_ref, wc_ref, wt_ref, bt_ref, out_ref):
    r = lat_ref.shape[0]
    wa = wa_ref[0, :]
    wb = wb_ref[0, :]
    wc = wc_ref[0, :]
    wt = wt_ref[0, :]
    bt = bt_ref[0, :]
    col = lax.broadcasted_iota(jnp.int32, (r, 64), 1)

    lat = lat_ref[:]
    lon = lon_ref[:]
    xloc = lat * wa[None, :] + lon * wb[None, :] + wc[None, :]
    out_ref[:, 0:64] = jnp.where(col >= 32, jnp.sin(xloc), xloc)

    out_ref[:, 64:128] = rows_ref[:]

    xa = ta_ref[:] * wt[None, :] + bt[None, :]
    out_ref[:, 128:192] = jnp.where(col > 0, jnp.sin(xa), xa)
    xd = td_ref[:] * wt[None, :] + bt[None, :]
    out_ref[:, 192:256] = jnp.where(col > 0, jnp.sin(xd), xd)


def _tc_encode(lat, lon, ta, td, rows, wa, wb, wc, wt, bt, n, interpret=False):
    r = _ROWS_BLK
    col_spec = pl.BlockSpec((r, 1), lambda i: (i, 0))
    wspec = pl.BlockSpec((1, 64), lambda i: (0, 0))
    return pl.pallas_call(
        _tc_encode_kernel,
        grid=(n // r,),
        in_specs=[col_spec, col_spec, col_spec, col_spec,
                  pl.BlockSpec((r, 64), lambda i: (i, 0)),
                  wspec, wspec, wspec, wspec, wspec],
        out_specs=pl.BlockSpec((r, 256), lambda i: (i, 0)),
        out_shape=jax.ShapeDtypeStruct((n, 256), jnp.float32),
        interpret=interpret,
    )(lat, lon, ta, td, rows, wa, wb, wc, wt, bt)


def kernel(region_id, latitude, longitude, arrival_time, departure_time,
           poly_W_lin, poly_b_lin, poly_W_per, poly_b_per,
           t2v_w0, t2v_b0, t2v_W, t2v_B, table):
    b, s = region_id.shape
    n = b * s
    d = table.shape[1]

    rows = _sc_gather(table, region_id.reshape(1, n), n, d)

    lat = latitude.reshape(n, 1)
    lon = longitude.reshape(n, 1)
    ta = arrival_time.reshape(n, 1)
    td = departure_time.reshape(n, 1)
    wa = jnp.concatenate([poly_W_lin[0], poly_W_per[0]]).reshape(1, 64)
    wb = jnp.concatenate([poly_W_lin[1], poly_W_per[1]]).reshape(1, 64)
    wc = jnp.concatenate([poly_b_lin, poly_b_per]).reshape(1, 64)
    wt = jnp.concatenate([t2v_w0, t2v_W]).reshape(1, 64)
    bt = jnp.concatenate([t2v_b0, t2v_B]).reshape(1, 64)

    out = _tc_encode(lat, lon, ta, td, rows, wa, wb, wc, wt, bt, n)
    return out.reshape(b, s, 4 * d)


# speedup vs baseline: 2.1572x; 1.6070x over previous
import functools

import jax
import jax.numpy as jnp
from jax import lax
from jax.experimental import pallas as pl
from jax.experimental.pallas import tpu as pltpu
from jax.experimental.pallas import tpu_sc as plsc

_PAD = 0
_GW = 128
_ROWS_BLK = 1024


def _sc_gather(table, idx2d, n, d):
    mesh = plsc.VectorSubcoreMesh(core_axis_name="core",
                                  subcore_axis_name="subcore")

    @functools.partial(
        pl.kernel,
        out_type=jax.ShapeDtypeStruct((n, d), table.dtype),
        mesh=mesh,
        compiler_params=pltpu.CompilerParams(use_tc_tiling_on_sc=False),
    )
    def k(tbl_hbm, i_hbm, o_hbm):
        def body(i_vmem, o_vmem):
            pltpu.sync_copy(tbl_hbm.at[i_vmem.at[0]], o_vmem)

        pltpu.emit_pipeline(
            body,
            grid=(n // _GW,),
            in_specs=[pl.BlockSpec((1, _GW), index_map=lambda i: (0, i))],
            out_specs=[pl.BlockSpec((_GW, d), index_map=lambda i: (i, 0))],
            core_axis_name=("core", "subcore"),
            dimension_semantics=(pltpu.PARALLEL,),
        )(i_hbm, o_hbm)

    return k(table, idx2d)


_PI_A = 3.140625
_PI_B = 9.67025756835937500e-04
_PI_C = 6.27711415290832519e-07
_PI_D = 1.21542012565534205e-10
_S3 = -0.166666597127914428710938
_S5 = 0.00833307858556509017944336
_S7 = -0.000198106907191686332226
_S9 = 0.0000026083159809786593541503


def _fast_sin(x):
    k = jnp.round(x * (1.0 / jnp.pi))
    r = x - k * _PI_A
    r = r - k * _PI_B
    r = r - k * _PI_C
    r = r - k * _PI_D
    s = r * r
    p = _S7 + s * _S9
    p = _S5 + s * p
    p = _S3 + s * p
    p = r + r * (s * p)
    odd = (k.astype(jnp.int32) & 1) == 1
    return jnp.where(odd, -p, p)


def _tc_encode_kernel(lat_ref, lon_ref, ta_ref, td_ref, rows_ref,
                      wa_ref, wb_ref, wc_ref, wt_ref, bt_ref, out_ref):
    r = lat_ref.shape[0]
    wa = wa_ref[0, :]
    wb = wb_ref[0, :]
    wc = wc_ref[0, :]
    wt = wt_ref[0, :]
    bt = bt_ref[0, :]
    col = lax.broadcasted_iota(jnp.int32, (r, 64), 1)

    lat = lat_ref[:]
    lon = lon_ref[:]
    xloc = lat * wa[None, :] + lon * wb[None, :] + wc[None, :]
    out_ref[:, 0:64] = jnp.where(col >= 32, _fast_sin(xloc), xloc)

    out_ref[:, 64:128] = rows_ref[:]

    xa = ta_ref[:] * wt[None, :] + bt[None, :]
    out_ref[:, 128:192] = jnp.where(col > 0, _fast_sin(xa), xa)
    xd = td_ref[:] * wt[None, :] + bt[None, :]
    out_ref[:, 192:256] = jnp.where(col > 0, _fast_sin(xd), xd)


def _tc_encode(lat, lon, ta, td, rows, wa, wb, wc, wt, bt, n, interpret=False):
    r = _ROWS_BLK
    col_spec = pl.BlockSpec((r, 1), lambda i: (i, 0))
    wspec = pl.BlockSpec((1, 64), lambda i: (0, 0))
    return pl.pallas_call(
        _tc_encode_kernel,
        grid=(n // r,),
        in_specs=[col_spec, col_spec, col_spec, col_spec,
                  pl.BlockSpec((r, 64), lambda i: (i, 0)),
                  wspec, wspec, wspec, wspec, wspec],
        out_specs=pl.BlockSpec((r, 256), lambda i: (i, 0)),
        out_shape=jax.ShapeDtypeStruct((n, 256), jnp.float32),
        interpret=interpret,
    )(lat, lon, ta, td, rows, wa, wb, wc, wt, bt)


def kernel(region_id, latitude, longitude, arrival_time, departure_time,
           poly_W_lin, poly_b_lin, poly_W_per, poly_b_per,
           t2v_w0, t2v_b0, t2v_W, t2v_B, table):
    b, s = region_id.shape
    n = b * s
    d = table.shape[1]

    rows = _sc_gather(table, region_id.reshape(1, n), n, d)

    lat = latitude.reshape(n, 1)
    lon = longitude.reshape(n, 1)
    ta = arrival_time.reshape(n, 1)
    td = departure_time.reshape(n, 1)
    wa = jnp.concatenate([poly_W_lin[0], poly_W_per[0]]).reshape(1, 64)
    wb = jnp.concatenate([poly_W_lin[1], poly_W_per[1]]).reshape(1, 64)
    wc = jnp.concatenate([poly_b_lin, poly_b_per]).reshape(1, 64)
    wt = jnp.concatenate([t2v_w0, t2v_W]).reshape(1, 64)
    bt = jnp.concatenate([t2v_b0, t2v_B]).reshape(1, 64)

    out = _tc_encode(lat, lon, ta, td, rows, wa, wb, wc, wt, bt, n)
    return out.reshape(b, s, 4 * d)


# speedup vs baseline: 2.6724x; 1.2388x over previous
import functools

import jax
import jax.numpy as jnp
from jax import lax
from jax.experimental import pallas as pl
from jax.experimental.pallas import tpu as pltpu
from jax.experimental.pallas import tpu_sc as plsc

_PAD = 0
_GW = 128
_ROWS_BLK = 1024


def _sc_gather(table, idx2d, n, d):
    mesh = plsc.VectorSubcoreMesh(core_axis_name="core",
                                  subcore_axis_name="subcore")

    @functools.partial(
        pl.kernel,
        out_type=jax.ShapeDtypeStruct((n, d), table.dtype),
        mesh=mesh,
        compiler_params=pltpu.CompilerParams(use_tc_tiling_on_sc=False),
    )
    def k(tbl_hbm, i_hbm, o_hbm):
        def body(i_vmem, o_vmem):
            pltpu.sync_copy(tbl_hbm.at[i_vmem.at[0]], o_vmem)

        pltpu.emit_pipeline(
            body,
            grid=(n // _GW,),
            in_specs=[pl.BlockSpec((1, _GW), index_map=lambda i: (0, i))],
            out_specs=[pl.BlockSpec((_GW, d), index_map=lambda i: (i, 0))],
            core_axis_name=("core", "subcore"),
            dimension_semantics=(pltpu.PARALLEL,),
        )(i_hbm, o_hbm)

    return k(table, idx2d)


_PI_A = 3.140625
_PI_B = 9.67025756835937500e-04
_PI_C = 6.27711415290832519e-07
_PI_D = 1.21542012565534205e-10
_S3 = -0.166666597127914428710938
_S5 = 0.00833307858556509017944336
_S7 = -0.000198106907191686332226
_S9 = 0.0000026083159809786593541503


def _fast_sin(x):
    k = jnp.round(x * (1.0 / jnp.pi))
    r = x - k * _PI_A
    r = r - k * _PI_B
    r = r - k * _PI_C
    r = r - k * _PI_D
    s = r * r
    p = _S7 + s * _S9
    p = _S5 + s * p
    p = _S3 + s * p
    p = r + r * (s * p)
    odd = (k.astype(jnp.int32) & 1) == 1
    return jnp.where(odd, -p, p)


def _tc_encode_kernel(lat_ref, lon_ref, ta_ref, td_ref, rows_ref,
                      wa_ref, wb_ref, wc_ref, wt_ref, bt_ref, out_ref):
    g = _ROWS_BLK // 128
    lane = lax.broadcasted_iota(jnp.int32, (128, 128), 1)
    wa = wa_ref[0, :]
    wb = wb_ref[0, :]
    wc = wc_ref[0, :]
    wt = wt_ref[0, :]
    bt = bt_ref[0, :]
    for j in range(g):
        rs = slice(j * 128, (j + 1) * 128)
        lat = jnp.transpose(lat_ref[0, j:j + 1, :], (1, 0))
        lon = jnp.transpose(lon_ref[0, j:j + 1, :], (1, 0))
        ta = jnp.transpose(ta_ref[0, j:j + 1, :], (1, 0))
        td = jnp.transpose(td_ref[0, j:j + 1, :], (1, 0))

        x0 = lat * wa + lon * wb + wc
        t0 = jnp.where(lane >= 32, _fast_sin(x0), x0)
        out_ref[rs, 0:64] = t0[:, 0:64]
        out_ref[rs, 64:128] = rows_ref[rs, :]

        tsel = jnp.where(lane < 64, jnp.broadcast_to(ta, (128, 128)),
                         jnp.broadcast_to(td, (128, 128)))
        x1 = tsel * wt + bt
        out_ref[rs, 128:256] = jnp.where((lane & 63) == 0, x1, _fast_sin(x1))


def _tc_encode(lat, lon, ta, td, rows, wa, wb, wc, wt, bt, n, interpret=False):
    r = _ROWS_BLK
    g = r // 128
    scal_spec = pl.BlockSpec((1, g, 128), lambda i: (i, 0, 0))
    wspec = pl.BlockSpec((1, 128), lambda i: (0, 0))
    return pl.pallas_call(
        _tc_encode_kernel,
        grid=(n // r,),
        in_specs=[scal_spec, scal_spec, scal_spec, scal_spec,
                  pl.BlockSpec((r, 64), lambda i: (i, 0)),
                  wspec, wspec, wspec, wspec, wspec],
        out_specs=pl.BlockSpec((r, 256), lambda i: (i, 0)),
        out_shape=jax.ShapeDtypeStruct((n, 256), jnp.float32),
        interpret=interpret,
    )(lat, lon, ta, td, rows, wa, wb, wc, wt, bt)


def kernel(region_id, latitude, longitude, arrival_time, departure_time,
           poly_W_lin, poly_b_lin, poly_W_per, poly_b_per,
           t2v_w0, t2v_b0, t2v_W, t2v_B, table):
    b, s = region_id.shape
    n = b * s
    d = table.shape[1]

    rows = _sc_gather(table, region_id.reshape(1, n), n, d)

    g = _ROWS_BLK // 128
    lat = latitude.reshape(n // _ROWS_BLK, g, 128)
    lon = longitude.reshape(n // _ROWS_BLK, g, 128)
    ta = arrival_time.reshape(n // _ROWS_BLK, g, 128)
    td = departure_time.reshape(n // _ROWS_BLK, g, 128)
    zeros64 = jnp.zeros((64,), jnp.float32)
    wa = jnp.concatenate([poly_W_lin[0], poly_W_per[0], zeros64]).reshape(1, 128)
    wb = jnp.concatenate([poly_W_lin[1], poly_W_per[1], zeros64]).reshape(1, 128)
    wc = jnp.concatenate([poly_b_lin, poly_b_per, zeros64]).reshape(1, 128)
    wt = jnp.concatenate([t2v_w0, t2v_W, t2v_w0, t2v_W]).reshape(1, 128)
    bt = jnp.concatenate([t2v_b0, t2v_B, t2v_b0, t2v_B]).reshape(1, 128)

    out = _tc_encode(lat, lon, ta, td, rows, wa, wb, wc, wt, bt, n)
    return out.reshape(b, s, 4 * d)


# speedup vs baseline: 2.7912x; 1.0445x over previous
import functools

import jax
import jax.numpy as jnp
from jax import lax
from jax.experimental import pallas as pl
from jax.experimental.pallas import tpu as pltpu
from jax.experimental.pallas import tpu_sc as plsc

_PAD = 0
_GW = 128
_ROWS_BLK = 1024


def _sc_gather(table2, idx2d, n):
    mesh = plsc.VectorSubcoreMesh(core_axis_name="core",
                                  subcore_axis_name="subcore")

    @functools.partial(
        pl.kernel,
        out_type=jax.ShapeDtypeStruct((n, 128), table2.dtype),
        mesh=mesh,
    )
    def k(tbl_hbm, i_hbm, o_hbm):
        def body(i_vmem, o_vmem):
            pltpu.sync_copy(tbl_hbm.at[i_vmem.at[0]], o_vmem)

        pltpu.emit_pipeline(
            body,
            grid=(n // _GW,),
            in_specs=[pl.BlockSpec((1, _GW), index_map=lambda i: (0, i))],
            out_specs=[pl.BlockSpec((_GW, 128), index_map=lambda i: (i, 0))],
            core_axis_name=("core", "subcore"),
            dimension_semantics=(pltpu.PARALLEL,),
        )(i_hbm, o_hbm)

    return k(table2, idx2d)


_PI_A = 3.140625
_PI_B = 9.67025756835937500e-04
_PI_C = 6.27711415290832519e-07
_PI_D = 1.21542012565534205e-10
_S3 = -0.166666597127914428710938
_S5 = 0.00833307858556509017944336
_S7 = -0.000198106907191686332226
_S9 = 0.0000026083159809786593541503


def _fast_sin(x):
    k = jnp.round(x * (1.0 / jnp.pi))
    r = x - k * _PI_A
    r = r - k * _PI_B
    r = r - k * _PI_C
    r = r - k * _PI_D
    s = r * r
    p = _S7 + s * _S9
    p = _S5 + s * p
    p = _S3 + s * p
    p = r + r * (s * p)
    odd = (k.astype(jnp.int32) & 1) == 1
    return jnp.where(odd, -p, p)


def _tc_encode_kernel(lat_ref, lon_ref, ta_ref, td_ref, rows_ref,
                      wa_ref, wb_ref, wc_ref, wt_ref, bt_ref, out_ref):
    g = _ROWS_BLK // 128
    lane = lax.broadcasted_iota(jnp.int32, (128, 128), 1)
    wa = wa_ref[0, :]
    wb = wb_ref[0, :]
    wc = wc_ref[0, :]
    wt = wt_ref[0, :]
    bt = bt_ref[0, :]
    for j in range(g):
        rs = slice(j * 128, (j + 1) * 128)
        lat = jnp.transpose(lat_ref[0, j:j + 1, :], (1, 0))
        lon = jnp.transpose(lon_ref[0, j:j + 1, :], (1, 0))
        ta = jnp.transpose(ta_ref[0, j:j + 1, :], (1, 0))
        td = jnp.transpose(td_ref[0, j:j + 1, :], (1, 0))

        x0 = lat * wa + lon * wb + wc
        t0 = jnp.where(lane >= 32, _fast_sin(x0), x0)
        out_ref[rs, 0:64] = t0[:, 0:64]
        out_ref[rs, 64:128] = rows_ref[rs, 0:64]

        tsel = jnp.where(lane < 64, jnp.broadcast_to(ta, (128, 128)),
                         jnp.broadcast_to(td, (128, 128)))
        x1 = tsel * wt + bt
        out_ref[rs, 128:256] = jnp.where((lane & 63) == 0, x1, _fast_sin(x1))


def _tc_encode(lat, lon, ta, td, rows, wa, wb, wc, wt, bt, n, interpret=False):
    r = _ROWS_BLK
    g = r // 128
    scal_spec = pl.BlockSpec((1, g, 128), lambda i: (i, 0, 0))
    wspec = pl.BlockSpec((1, 128), lambda i: (0, 0))
    return pl.pallas_call(
        _tc_encode_kernel,
        grid=(n // r,),
        in_specs=[scal_spec, scal_spec, scal_spec, scal_spec,
                  pl.BlockSpec((r, 128), lambda i: (i, 0)),
                  wspec, wspec, wspec, wspec, wspec],
        out_specs=pl.BlockSpec((r, 256), lambda i: (i, 0)),
        out_shape=jax.ShapeDtypeStruct((n, 256), jnp.float32),
        interpret=interpret,
    )(lat, lon, ta, td, rows, wa, wb, wc, wt, bt)


def kernel(region_id, latitude, longitude, arrival_time, departure_time,
           poly_W_lin, poly_b_lin, poly_W_per, poly_b_per,
           t2v_w0, t2v_b0, t2v_W, t2v_B, table):
    b, s = region_id.shape
    n = b * s
    d = table.shape[1]

    table2 = jnp.concatenate([table, table], axis=1)
    rows = _sc_gather(table2, region_id.reshape(1, n), n)

    g = _ROWS_BLK // 128
    lat = latitude.reshape(n // _ROWS_BLK, g, 128)
    lon = longitude.reshape(n // _ROWS_BLK, g, 128)
    ta = arrival_time.reshape(n // _ROWS_BLK, g, 128)
    td = departure_time.reshape(n // _ROWS_BLK, g, 128)
    zeros64 = jnp.zeros((64,), jnp.float32)
    wa = jnp.concatenate([poly_W_lin[0], poly_W_per[0], zeros64]).reshape(1, 128)
    wb = jnp.concatenate([poly_W_lin[1], poly_W_per[1], zeros64]).reshape(1, 128)
    wc = jnp.concatenate([poly_b_lin, poly_b_per, zeros64]).reshape(1, 128)
    wt = jnp.concatenate([t2v_w0, t2v_W, t2v_w0, t2v_W]).reshape(1, 128)
    bt = jnp.concatenate([t2v_b0, t2v_B, t2v_b0, t2v_B]).reshape(1, 128)

    out = _tc_encode(lat, lon, ta, td, rows, wa, wb, wc, wt, bt, n)
    return out.reshape(b, s, 4 * d)


# speedup vs baseline: 2.7941x; 1.0010x over previous
import functools

import jax
import jax.numpy as jnp
from jax import lax
from jax.experimental import pallas as pl
from jax.experimental.pallas import tpu as pltpu
from jax.experimental.pallas import tpu_sc as plsc

_PAD = 0
_GW = 128
_ROWS_BLK = 1024


def _sc_gather(table2, idx2d, n):
    mesh = plsc.VectorSubcoreMesh(core_axis_name="core",
                                  subcore_axis_name="subcore")

    @functools.partial(
        pl.kernel,
        out_type=jax.ShapeDtypeStruct((n, 128), table2.dtype),
        mesh=mesh,
        compiler_params=pltpu.CompilerParams(use_tc_tiling_on_sc=True),
    )
    def k(tbl_hbm, i_hbm, o_hbm):
        def body(i_vmem, o_vmem):
            pltpu.sync_copy(tbl_hbm.at[i_vmem.at[0]], o_vmem)

        pltpu.emit_pipeline(
            body,
            grid=(n // _GW,),
            in_specs=[pl.BlockSpec((1, _GW), index_map=lambda i: (0, i))],
            out_specs=[pl.BlockSpec((_GW, 128), index_map=lambda i: (i, 0))],
            core_axis_name=("core", "subcore"),
            dimension_semantics=(pltpu.PARALLEL,),
        )(i_hbm, o_hbm)

    return k(table2, idx2d)


_PI_A = 3.140625
_PI_B = 9.67025756835937500e-04
_PI_C = 6.27711415290832519e-07
_PI_D = 1.21542012565534205e-10
_S3 = -0.166666597127914428710938
_S5 = 0.00833307858556509017944336
_S7 = -0.000198106907191686332226
_S9 = 0.0000026083159809786593541503


def _fast_sin(x):
    k = jnp.round(x * (1.0 / jnp.pi))
    r = x - k * _PI_A
    r = r - k * _PI_B
    r = r - k * _PI_C
    r = r - k * _PI_D
    s = r * r
    p = _S7 + s * _S9
    p = _S5 + s * p
    p = _S3 + s * p
    p = r + r * (s * p)
    odd = (k.astype(jnp.int32) & 1) == 1
    return jnp.where(odd, -p, p)


def _tc_encode_kernel(lat_ref, lon_ref, ta_ref, td_ref, rows_ref,
                      wa_ref, wb_ref, wc_ref, wt_ref, bt_ref, out_ref):
    g = _ROWS_BLK // 128
    lane = lax.broadcasted_iota(jnp.int32, (128, 128), 1)
    wa = wa_ref[0, :]
    wb = wb_ref[0, :]
    wc = wc_ref[0, :]
    wt = wt_ref[0, :]
    bt = bt_ref[0, :]
    for j in range(g):
        rs = slice(j * 128, (j + 1) * 128)
        lat = jnp.transpose(lat_ref[0, j:j + 1, :], (1, 0))
        lon = jnp.transpose(lon_ref[0, j:j + 1, :], (1, 0))
        ta = jnp.transpose(ta_ref[0, j:j + 1, :], (1, 0))
        td = jnp.transpose(td_ref[0, j:j + 1, :], (1, 0))

        x0 = lat * wa + lon * wb + wc
        t0 = jnp.where(lane >= 32, _fast_sin(x0), x0)
        out_ref[rs, 0:64] = t0[:, 0:64]
        out_ref[rs, 64:128] = rows_ref[rs, 0:64]

        tsel = jnp.where(lane < 64, jnp.broadcast_to(ta, (128, 128)),
                         jnp.broadcast_to(td, (128, 128)))
        x1 = tsel * wt + bt
        out_ref[rs, 128:256] = jnp.where((lane & 63) == 0, x1, _fast_sin(x1))


def _tc_encode(lat, lon, ta, td, rows, wa, wb, wc, wt, bt, n, interpret=False):
    r = _ROWS_BLK
    g = r // 128
    scal_spec = pl.BlockSpec((1, g, 128), lambda i: (i, 0, 0))
    wspec = pl.BlockSpec((1, 128), lambda i: (0, 0))
    return pl.pallas_call(
        _tc_encode_kernel,
        grid=(n // r,),
        in_specs=[scal_spec, scal_spec, scal_spec, scal_spec,
                  pl.BlockSpec((r, 128), lambda i: (i, 0)),
                  wspec, wspec, wspec, wspec, wspec],
        out_specs=pl.BlockSpec((r, 256), lambda i: (i, 0)),
        out_shape=jax.ShapeDtypeStruct((n, 256), jnp.float32),
        interpret=interpret,
    )(lat, lon, ta, td, rows, wa, wb, wc, wt, bt)


def kernel(region_id, latitude, longitude, arrival_time, departure_time,
           poly_W_lin, poly_b_lin, poly_W_per, poly_b_per,
           t2v_w0, t2v_b0, t2v_W, t2v_B, table):
    b, s = region_id.shape
    n = b * s
    d = table.shape[1]

    table2 = jnp.concatenate([table, table], axis=1)
    rows = _sc_gather(table2, region_id.reshape(1, n), n)

    g = _ROWS_BLK // 128
    lat = latitude.reshape(n // _ROWS_BLK, g, 128)
    lon = longitude.reshape(n // _ROWS_BLK, g, 128)
    ta = arrival_time.reshape(n // _ROWS_BLK, g, 128)
    td = departure_time.reshape(n // _ROWS_BLK, g, 128)
    zeros64 = jnp.zeros((64,), jnp.float32)
    wa = jnp.concatenate([poly_W_lin[0], poly_W_per[0], zeros64]).reshape(1, 128)
    wb = jnp.concatenate([poly_W_lin[1], poly_W_per[1], zeros64]).reshape(1, 128)
    wc = jnp.concatenate([poly_b_lin, poly_b_per, zeros64]).reshape(1, 128)
    wt = jnp.concatenate([t2v_w0, t2v_W, t2v_w0, t2v_W]).reshape(1, 128)
    bt = jnp.concatenate([t2v_b0, t2v_B, t2v_b0, t2v_B]).reshape(1, 128)

    out = _tc_encode(lat, lon, ta, td, rows, wa, wb, wc, wt, bt, n)
    return out.reshape(b, s, 4 * d)


# speedup vs baseline: 4.6840x; 1.6764x over previous
import functools

import jax
import jax.numpy as jnp
from jax import lax
from jax.experimental import pallas as pl
from jax.experimental.pallas import tpu as pltpu
from jax.experimental.pallas import tpu_sc as plsc

_PAD = 0
_GW = 128
_ROWS_BLK = 1024


def _sc_gather(table2, idx2d, n):
    mesh = plsc.VectorSubcoreMesh(core_axis_name="core",
                                  subcore_axis_name="subcore")

    @functools.partial(
        pl.kernel,
        out_type=jax.ShapeDtypeStruct((n, 128), table2.dtype),
        mesh=mesh,
        compiler_params=pltpu.CompilerParams(use_tc_tiling_on_sc=True),
    )
    def k(tbl_hbm, i_hbm, o_hbm):
        def body(i_vmem, o_vmem):
            pltpu.sync_copy(tbl_hbm.at[i_vmem.at[0]], o_vmem)

        pltpu.emit_pipeline(
            body,
            grid=(n // _GW,),
            in_specs=[pl.BlockSpec((1, _GW), index_map=lambda i: (0, i))],
            out_specs=[pl.BlockSpec((_GW, 128), index_map=lambda i: (i, 0))],
            core_axis_name=("core", "subcore"),
            dimension_semantics=(pltpu.PARALLEL,),
        )(i_hbm, o_hbm)

    return k(table2, idx2d)


_PI_A = 3.140625
_PI_B = 9.67025756835937500e-04
_PI_C = 6.27711415290832519e-07
_PI_D = 1.21542012565534205e-10
_S3 = -0.166666597127914428710938
_S5 = 0.00833307858556509017944336
_S7 = -0.000198106907191686332226
_S9 = 0.0000026083159809786593541503


def _fast_sin(x):
    k = jnp.round(x * (1.0 / jnp.pi))
    r = x - k * _PI_A
    r = r - k * _PI_B
    r = r - k * _PI_C
    r = r - k * _PI_D
    s = r * r
    p = _S7 + s * _S9
    p = _S5 + s * p
    p = _S3 + s * p
    p = r + r * (s * p)
    odd = (k.astype(jnp.int32) & 1) == 1
    return jnp.where(odd, -p, p)


def _tc_encode_kernel(lat_ref, lon_ref, ta_ref, td_ref, rows_ref,
                      wa_ref, wb_ref, wc_ref, wt_ref, bt_ref, out_ref):
    g = _ROWS_BLK // 128
    lane = lax.broadcasted_iota(jnp.int32, (128, 128), 1)
    wa = wa_ref[0, :]
    wb = wb_ref[0, :]
    wc = wc_ref[0, :]
    wt = wt_ref[0, :]
    bt = bt_ref[0, :]
    for j in range(g):
        rs = slice(j * 128, (j + 1) * 128)
        lat = jnp.transpose(lat_ref[0, j:j + 1, :], (1, 0))
        lon = jnp.transpose(lon_ref[0, j:j + 1, :], (1, 0))
        ta = jnp.transpose(ta_ref[0, j:j + 1, :], (1, 0))
        td = jnp.transpose(td_ref[0, j:j + 1, :], (1, 0))

        x0 = lat * wa + lon * wb + wc
        t0 = jnp.where(lane >= 32, _fast_sin(x0), x0)
        out_ref[rs, 0:64] = t0[:, 0:64]
        out_ref[rs, 64:128] = rows_ref[rs, 0:64]

        tsel = jnp.where(lane < 64, jnp.broadcast_to(ta, (128, 128)),
                         jnp.broadcast_to(td, (128, 128)))
        x1 = tsel * wt + bt
        out_ref[rs, 128:256] = jnp.where((lane & 63) == 0, x1, _fast_sin(x1))


def _tc_encode(lat, lon, ta, td, rows, wa, wb, wc, wt, bt, n, interpret=False):
    r = _ROWS_BLK
    g = r // 128
    scal_spec = pl.BlockSpec((1, g, 128), lambda i: (i, 0, 0))
    wspec = pl.BlockSpec((1, 128), lambda i: (0, 0))
    return pl.pallas_call(
        _tc_encode_kernel,
        grid=(n // r,),
        in_specs=[scal_spec, scal_spec, scal_spec, scal_spec,
                  pl.BlockSpec((r, 128), lambda i: (i, 0)),
                  wspec, wspec, wspec, wspec, wspec],
        out_specs=pl.BlockSpec((r, 256), lambda i: (i, 0)),
        out_shape=jax.ShapeDtypeStruct((n, 256), jnp.float32),
        interpret=interpret,
    )(lat, lon, ta, td, rows, wa, wb, wc, wt, bt)


def kernel(region_id, latitude, longitude, arrival_time, departure_time,
           poly_W_lin, poly_b_lin, poly_W_per, poly_b_per,
           t2v_w0, t2v_b0, t2v_W, t2v_B, table):
    b, s = region_id.shape
    n = b * s
    d = table.shape[1]

    table2 = jnp.concatenate([table, table], axis=1)
    rows = _sc_gather(table2, region_id.T.reshape(1, n), n)

    g = _ROWS_BLK // 128
    lat = latitude.T.reshape(n // _ROWS_BLK, g, 128)
    lon = longitude.T.reshape(n // _ROWS_BLK, g, 128)
    ta = arrival_time.T.reshape(n // _ROWS_BLK, g, 128)
    td = departure_time.T.reshape(n // _ROWS_BLK, g, 128)
    zeros64 = jnp.zeros((64,), jnp.float32)
    wa = jnp.concatenate([poly_W_lin[0], poly_W_per[0], zeros64]).reshape(1, 128)
    wb = jnp.concatenate([poly_W_lin[1], poly_W_per[1], zeros64]).reshape(1, 128)
    wc = jnp.concatenate([poly_b_lin, poly_b_per, zeros64]).reshape(1, 128)
    wt = jnp.concatenate([t2v_w0, t2v_W, t2v_w0, t2v_W]).reshape(1, 128)
    bt = jnp.concatenate([t2v_b0, t2v_B, t2v_b0, t2v_B]).reshape(1, 128)

    out = _tc_encode(lat, lon, ta, td, rows, wa, wb, wc, wt, bt, n)
    return jnp.transpose(out.reshape(s, b, 4 * d), (1, 0, 2))


# speedup vs baseline: 5.1614x; 1.1019x over previous
import functools

import jax
import jax.numpy as jnp
from jax import lax
from jax.experimental import pallas as pl
from jax.experimental.pallas import tpu as pltpu
from jax.experimental.pallas import tpu_sc as plsc

_PAD = 0
_GW = 128
_ROWS_BLK = 1024


def _sc_gather(table2, idx2d, n):
    mesh = plsc.VectorSubcoreMesh(core_axis_name="core",
                                  subcore_axis_name="subcore")

    @functools.partial(
        pl.kernel,
        out_type=jax.ShapeDtypeStruct((n, 128), table2.dtype),
        mesh=mesh,
        compiler_params=pltpu.CompilerParams(use_tc_tiling_on_sc=True),
    )
    def k(tbl_hbm, i_hbm, o_hbm):
        def body(i_vmem, o_vmem):
            pltpu.sync_copy(tbl_hbm.at[i_vmem.at[0]], o_vmem)

        pltpu.emit_pipeline(
            body,
            grid=(n // _GW,),
            in_specs=[pl.BlockSpec((1, _GW), index_map=lambda i: (0, i))],
            out_specs=[pl.BlockSpec((_GW, 128), index_map=lambda i: (i, 0))],
            core_axis_name=("core", "subcore"),
            dimension_semantics=(pltpu.PARALLEL,),
        )(i_hbm, o_hbm)

    return k(table2, idx2d)


_PI_A = 3.140625
_PI_B = 9.67025756835937500e-04
_PI_C = 6.27711415290832519e-07
_PI_D = 1.21542012565534205e-10
_S3 = -0.166666597127914428710938
_S5 = 0.00833307858556509017944336
_S7 = -0.000198106907191686332226
_S9 = 0.0000026083159809786593541503


def _fast_sin(x):
    k = jnp.round(x * (1.0 / jnp.pi))
    r = x - k * _PI_A
    r = r - k * _PI_B
    r = r - k * _PI_C
    r = r - k * _PI_D
    s = r * r
    p = _S7 + s * _S9
    p = _S5 + s * p
    p = _S3 + s * p
    p = r + r * (s * p)
    sgn = jnp.left_shift(k.astype(jnp.int32), 31)
    return lax.bitcast_convert_type(
        lax.bitcast_convert_type(p, jnp.int32) ^ sgn, jnp.float32)


def _tc_encode_kernel(lat_ref, lon_ref, ta_ref, td_ref, rows_ref,
                      wa_ref, wb_ref, wc_ref, wt_ref, bt_ref, out_ref):
    g = _ROWS_BLK // 128
    lane = lax.broadcasted_iota(jnp.int32, (128, 128), 1)
    wa = wa_ref[0, :]
    wb = wb_ref[0, :]
    wc = wc_ref[0, :]
    wt = wt_ref[0, :]
    bt = bt_ref[0, :]
    left = lane < 64
    m_per = (lane & 63) >= 32
    m_lin = (lane & 63) == 0
    for j in range(g):
        rs = slice(j * 128, (j + 1) * 128)
        lat = jnp.transpose(lat_ref[0, j:j + 1, :], (1, 0))
        lon = jnp.transpose(lon_ref[0, j:j + 1, :], (1, 0))
        ta = jnp.transpose(ta_ref[0, j:j + 1, :], (1, 0))
        td = jnp.transpose(td_ref[0, j:j + 1, :], (1, 0))

        x0 = lat * wa + lon * wb + wc
        t0 = jnp.where(m_per, _fast_sin(x0), x0)
        out_ref[rs, 0:64] = t0[:, 0:64]
        out_ref[rs, 64:128] = rows_ref[rs, 0:64]

        tsel = jnp.where(left, jnp.broadcast_to(ta, (128, 128)),
                         jnp.broadcast_to(td, (128, 128)))
        x1 = tsel * wt + bt
        out_ref[rs, 128:256] = jnp.where(m_lin, x1, _fast_sin(x1))


def _tc_encode(lat, lon, ta, td, rows, wa, wb, wc, wt, bt, n,
               nblocks=None, ofs=0, prev=None, interpret=False):
    r = _ROWS_BLK
    g = r // 128
    if nblocks is None:
        nblocks = n // r
    scal_spec = pl.BlockSpec((1, g, 128), lambda i: (i + ofs, 0, 0))
    wspec = pl.BlockSpec((1, 128), lambda i: (0, 0))
    in_specs = [scal_spec, scal_spec, scal_spec, scal_spec,
                pl.BlockSpec((r, 128), lambda i: (i, 0)),
                wspec, wspec, wspec, wspec, wspec]
    args = (lat, lon, ta, td, rows, wa, wb, wc, wt, bt)
    out_spec = pl.BlockSpec((r, 256), lambda i: (i + ofs, 0))
    out_shape = jax.ShapeDtypeStruct((n, 256), jnp.float32)
    if prev is None:
        return pl.pallas_call(
            _tc_encode_kernel, grid=(nblocks,),
            in_specs=in_specs, out_specs=out_spec, out_shape=out_shape,
            interpret=interpret,
        )(*args)

    def k2(prev_ref, *refs):
        _tc_encode_kernel(*refs)

    return pl.pallas_call(
        k2, grid=(nblocks,),
        in_specs=[pl.BlockSpec(memory_space=pl.ANY)] + in_specs,
        out_specs=out_spec, out_shape=out_shape,
        input_output_aliases={0: 0},
        interpret=interpret,
    )(prev, *args)


def kernel(region_id, latitude, longitude, arrival_time, departure_time,
           poly_W_lin, poly_b_lin, poly_W_per, poly_b_per,
           t2v_w0, t2v_b0, t2v_W, t2v_B, table):
    b, s = region_id.shape
    n = b * s
    d = table.shape[1]

    table2 = jnp.concatenate([table, table], axis=1)
    idx_t = region_id.T.reshape(1, n)
    half = n // 2
    rows_a = _sc_gather(table2, idx_t[:, :half], half)
    rows_b = _sc_gather(table2, idx_t[:, half:], half)

    g = _ROWS_BLK // 128
    lat = latitude.T.reshape(n // _ROWS_BLK, g, 128)
    lon = longitude.T.reshape(n // _ROWS_BLK, g, 128)
    ta = arrival_time.T.reshape(n // _ROWS_BLK, g, 128)
    td = departure_time.T.reshape(n // _ROWS_BLK, g, 128)
    wa = jnp.concatenate([poly_W_lin[0], poly_W_per[0],
                          poly_W_lin[0], poly_W_per[0]]).reshape(1, 128)
    wb = jnp.concatenate([poly_W_lin[1], poly_W_per[1],
                          poly_W_lin[1], poly_W_per[1]]).reshape(1, 128)
    wc = jnp.concatenate([poly_b_lin, poly_b_per,
                          poly_b_lin, poly_b_per]).reshape(1, 128)
    wt = jnp.concatenate([t2v_w0, t2v_W, t2v_w0, t2v_W]).reshape(1, 128)
    bt = jnp.concatenate([t2v_b0, t2v_B, t2v_b0, t2v_B]).reshape(1, 128)

    nb2 = n // _ROWS_BLK // 2
    out1 = _tc_encode(lat, lon, ta, td, rows_a, wa, wb, wc, wt, bt, n,
                      nblocks=nb2, ofs=0)
    out = _tc_encode(lat, lon, ta, td, rows_b, wa, wb, wc, wt, bt, n,
                     nblocks=nb2, ofs=nb2, prev=out1)
    return jnp.transpose(out.reshape(s, b, 4 * d), (1, 0, 2))
